# baseline (speedup 1.0000x reference)
import functools

import jax
import jax.numpy as jnp
from jax import lax
from jax.experimental import pallas as pl
from jax.experimental.pallas import tpu as pltpu
from jax.experimental.pallas import tpu_sc as plsc

N_NODES = 10000
D = 128
N_EDGES = 320000

NC = 2
NS = 16
NW = NC * NS
E_PER_W = N_EDGES // NW
CHUNK = 80
K = E_PER_W // CHUNK
STAGES = 5
SK = K // STAGES
NBUF = 3
SLAB = 624
ZCHUNK = 78


def _sc_partial_sums(x, row1d, col1d):
  mesh = plsc.VectorSubcoreMesh(core_axis_name="c", subcore_axis_name="s")

  @functools.partial(
      pl.kernel,
      out_type=jax.ShapeDtypeStruct((NC, N_NODES, D), jnp.float32),
      mesh=mesh,
      scratch_types=[
          pltpu.VMEM((SK, CHUNK), jnp.int32),
          pltpu.VMEM((SK, CHUNK), jnp.int32),
          pltpu.VMEM((SK, CHUNK), jnp.int32),
          pltpu.VMEM((SK, CHUNK), jnp.int32),
          pltpu.VMEM((CHUNK, D), jnp.float32),
          pltpu.VMEM((CHUNK, D), jnp.float32),
          pltpu.VMEM((CHUNK, D), jnp.float32),
          pltpu.VMEM_SHARED((N_NODES, D), jnp.float32),
          pltpu.SemaphoreType.DMA,
          pltpu.SemaphoreType.DMA,
          pltpu.SemaphoreType.DMA,
          pltpu.SemaphoreType.DMA,
      ],
  )
  def sc_kernel(x_hbm, row_hbm, col_hbm, out_hbm,
                cidxA, cidxB, ridxA, ridxB, buf0, buf1, buf2, acc,
                sem0, sem1, sem2, isem):
    c = lax.axis_index("c")
    s = lax.axis_index("s")
    w = c * NS + s
    ebase = w * E_PER_W

    def fire_idx_stage(h, cidx, ridx):
      off = ebase + h * SK * CHUNK
      for k in range(SK):
        pltpu.async_copy(col_hbm.at[pl.ds(off + k * CHUNK, CHUNK)],
                         cidx.at[k], isem)
        pltpu.async_copy(row_hbm.at[pl.ds(off + k * CHUNK, CHUNK)],
                         ridx.at[k], isem)

    def drain_idx_stage(h, cidx, ridx):
      off = ebase + h * SK * CHUNK
      for k in range(SK):
        pltpu.make_async_copy(col_hbm.at[pl.ds(off + k * CHUNK, CHUNK)],
                              cidx.at[k], isem).wait()
        pltpu.make_async_copy(row_hbm.at[pl.ds(off + k * CHUNK, CHUNK)],
                              ridx.at[k], isem).wait()

    idx_bufs = [(cidxA, ridxA), (cidxB, ridxB)]
    fire_idx_stage(0, cidxA, ridxA)

    def zero_row(i, carry):
      for j in range(D // 16):
        buf0[i, pl.ds(j * 16, 16)] = jnp.zeros((16,), jnp.float32)
      return carry
    lax.fori_loop(0, ZCHUNK, zero_row, 0)
    base = s * SLAB
    for j in range(SLAB // ZCHUNK):
      pltpu.sync_copy(buf0.at[pl.ds(0, ZCHUNK)],
                      acc.at[pl.ds(base + j * ZCHUNK, ZCHUNK)])

    @pl.when(s == NS - 1)
    def _():
      pltpu.sync_copy(buf0.at[pl.ds(0, 16)],
                      acc.at[pl.ds(NS * SLAB, 16)])

    plsc.subcore_barrier()

    bufs = (buf0, buf1, buf2)
    sems = (sem0, sem1, sem2)

    for h in range(STAGES):
      cidx, ridx = idx_bufs[h % 2]
      drain_idx_stage(h, cidx, ridx)
      pltpu.async_copy(x_hbm.at[cidx.at[0]], buf0, sem0)
      pltpu.async_copy(x_hbm.at[cidx.at[1]], buf1, sem1)
      if h + 1 < STAGES:
        ncidx, nridx = idx_bufs[(h + 1) % 2]
        fire_idx_stage(h + 1, ncidx, nridx)

      def step(k, b):
        pltpu.make_async_copy(x_hbm.at[cidx.at[k]], bufs[b], sems[b]).wait()

        @pl.when(k + 2 < SK)
        def _():
          b2 = (b + 2) % NBUF
          pltpu.async_copy(x_hbm.at[cidx.at[k + 2]], bufs[b2], sems[b2])

        pltpu.sync_copy(bufs[b], acc.at[ridx.at[k]], add=True)

      def body(i, carry):
        for u in range(NBUF):
          step(i * NBUF + u, u)
        return carry
      lax.fori_loop(0, SK // NBUF, body, 0)
      step(SK - 1, 0)

    plsc.subcore_barrier()
    pltpu.sync_copy(acc.at[pl.ds(base, SLAB)],
                    out_hbm.at[c, pl.ds(base, SLAB)])

    @pl.when(s == NS - 1)
    def _():
      pltpu.sync_copy(acc.at[pl.ds(NS * SLAB, 16)],
                      out_hbm.at[c, pl.ds(NS * SLAB, 16)])

  return sc_kernel(x, row1d, col1d)


def _tc_add(a, b):
  def add_kernel(a_ref, b_ref, o_ref):
    o_ref[...] = a_ref[...] + b_ref[...]

  block = pl.BlockSpec((1000, D), lambda i: (i, 0))
  return pl.pallas_call(
      add_kernel,
      grid=(N_NODES // 1000,),
      in_specs=[block, block],
      out_specs=block,
      out_shape=jax.ShapeDtypeStruct((N_NODES, D), jnp.float32),
  )(a, b)


@jax.jit
def kernel(x, edge_index):
  ei = edge_index.astype(jnp.int32)
  partials = _sc_partial_sums(x, ei[0], ei[1])
  return _tc_add(partials[0], partials[1])


# speedup vs baseline: 1.0083x; 1.0083x over previous
import functools

import jax
import jax.numpy as jnp
from jax import lax
from jax.experimental import pallas as pl
from jax.experimental.pallas import tpu as pltpu
from jax.experimental.pallas import tpu_sc as plsc

N_NODES = 10000
D = 128
N_EDGES = 320000

NC = 2
NS = 16
NW = NC * NS
E_PER_W = N_EDGES // NW
CHUNK = 80
K = E_PER_W // CHUNK
STAGES = 5
SK = K // STAGES
NBUF = 3
SLAB = 624
ZCHUNK = 78


def _sc_partial_sums(x, row4, col1d):
  mesh = plsc.VectorSubcoreMesh(core_axis_name="c", subcore_axis_name="s")

  @functools.partial(
      pl.kernel,
      out_type=jax.ShapeDtypeStruct((NC, N_NODES, D), jnp.float32),
      mesh=mesh,
      scratch_types=[
          pltpu.VMEM((E_PER_W,), jnp.int32),
          pltpu.VMEM((SK, CHUNK), jnp.int32),
          pltpu.VMEM((SK, CHUNK), jnp.int32),
          pltpu.VMEM((CHUNK, D), jnp.float32),
          pltpu.VMEM((CHUNK, D), jnp.float32),
          pltpu.VMEM((CHUNK, D), jnp.float32),
          pltpu.VMEM_SHARED((N_NODES, D), jnp.float32),
          pltpu.SemaphoreType.DMA,
          pltpu.SemaphoreType.DMA,
          pltpu.SemaphoreType.DMA,
          pltpu.SemaphoreType.DMA,
      ],
  )
  def sc_kernel(x_hbm, row_hbm, col_hbm, out_hbm,
                cidx, ridxA, ridxB, buf0, buf1, buf2, acc,
                sem0, sem1, sem2, isem):
    c = lax.axis_index("c")
    s = lax.axis_index("s")
    w = c * NS + s

    pltpu.async_copy(col_hbm.at[pl.ds(w * E_PER_W, E_PER_W)], cidx, isem)
    pltpu.async_copy(row_hbm.at[w, 0], ridxA, isem)

    def zero_row(i, carry):
      for j in range(D // 16):
        buf0[i, pl.ds(j * 16, 16)] = jnp.zeros((16,), jnp.float32)
      return carry
    lax.fori_loop(0, ZCHUNK, zero_row, 0)
    base = s * SLAB
    for j in range(SLAB // ZCHUNK):
      pltpu.sync_copy(buf0.at[pl.ds(0, ZCHUNK)],
                      acc.at[pl.ds(base + j * ZCHUNK, ZCHUNK)])

    @pl.when(s == NS - 1)
    def _():
      pltpu.sync_copy(buf0.at[pl.ds(0, 16)],
                      acc.at[pl.ds(NS * SLAB, 16)])

    pltpu.make_async_copy(col_hbm.at[pl.ds(w * E_PER_W, E_PER_W)],
                          cidx, isem).wait()
    pltpu.make_async_copy(row_hbm.at[w, 0], ridxA, isem).wait()
    plsc.subcore_barrier()

    bufs = (buf0, buf1, buf2)
    sems = (sem0, sem1, sem2)
    ridxs = (ridxA, ridxB)

    for h in range(STAGES):
      ridx = ridxs[h % 2]

      def gidx(k):
        return cidx.at[pl.ds((h * SK + k) * CHUNK, CHUNK)]

      pltpu.async_copy(x_hbm.at[gidx(0)], buf0, sem0)
      pltpu.async_copy(x_hbm.at[gidx(1)], buf1, sem1)
      if h + 1 < STAGES:
        pltpu.async_copy(row_hbm.at[w, h + 1], ridxs[(h + 1) % 2], isem)

      def step(k, b):
        pltpu.make_async_copy(x_hbm.at[gidx(k)], bufs[b], sems[b]).wait()

        @pl.when(k + 2 < SK)
        def _():
          b2 = (b + 2) % NBUF
          pltpu.async_copy(x_hbm.at[gidx(k + 2)], bufs[b2], sems[b2])

        pltpu.sync_copy(bufs[b], acc.at[ridx.at[k]], add=True)

      def body(i, carry):
        for u in range(NBUF):
          step(i * NBUF + u, u)
        return carry
      lax.fori_loop(0, SK // NBUF, body, 0)
      step(SK - 1, 0)

      if h + 1 < STAGES:
        pltpu.make_async_copy(row_hbm.at[w, h + 1],
                              ridxs[(h + 1) % 2], isem).wait()

    plsc.subcore_barrier()
    pltpu.sync_copy(acc.at[pl.ds(base, SLAB)],
                    out_hbm.at[c, pl.ds(base, SLAB)])

    @pl.when(s == NS - 1)
    def _():
      pltpu.sync_copy(acc.at[pl.ds(NS * SLAB, 16)],
                      out_hbm.at[c, pl.ds(NS * SLAB, 16)])

  return sc_kernel(x, row4, col1d)


def _tc_add(a, b):
  def add_kernel(a_ref, b_ref, o_ref):
    o_ref[...] = a_ref[...] + b_ref[...]

  block = pl.BlockSpec((1000, D), lambda i: (i, 0))
  return pl.pallas_call(
      add_kernel,
      grid=(N_NODES // 1000,),
      in_specs=[block, block],
      out_specs=block,
      out_shape=jax.ShapeDtypeStruct((N_NODES, D), jnp.float32),
  )(a, b)


@jax.jit
def kernel(x, edge_index):
  ei = edge_index.astype(jnp.int32)
  row4 = ei[0].reshape(NW, STAGES, SK, CHUNK)
  partials = _sc_partial_sums(x, row4, ei[1])
  return _tc_add(partials[0], partials[1])


# speedup vs baseline: 1.1710x; 1.1614x over previous
import functools

import jax
import jax.numpy as jnp
from jax import lax
from jax.experimental import pallas as pl
from jax.experimental.pallas import tpu as pltpu
from jax.experimental.pallas import tpu_sc as plsc

N_NODES = 10000
D = 128
N_EDGES = 320000

NC = 2
NS = 16
NW = NC * NS
E_PER_W = N_EDGES // NW
CHUNK = 80
K = E_PER_W // CHUNK
SK = 5
STAGES = K // SK
NBUF = 4
PERIOD = 20
SLAB = 624
ZCHUNK = 78


def _sc_partial_sums(x, ei5):
  mesh = plsc.VectorSubcoreMesh(core_axis_name="c", subcore_axis_name="s")

  @functools.partial(
      pl.kernel,
      out_type=jax.ShapeDtypeStruct((NC, N_NODES, D), jnp.float32),
      mesh=mesh,
      scratch_types=[
          pltpu.VMEM((SK, CHUNK), jnp.int32),
          pltpu.VMEM((SK, CHUNK), jnp.int32),
          pltpu.VMEM((SK, CHUNK), jnp.int32),
          pltpu.VMEM((SK, CHUNK), jnp.int32),
          pltpu.VMEM((CHUNK, D), jnp.float32),
          pltpu.VMEM((CHUNK, D), jnp.float32),
          pltpu.VMEM((CHUNK, D), jnp.float32),
          pltpu.VMEM((CHUNK, D), jnp.float32),
          pltpu.VMEM_SHARED((N_NODES, D), jnp.float32),
          pltpu.SemaphoreType.DMA,
          pltpu.SemaphoreType.DMA,
          pltpu.SemaphoreType.DMA,
          pltpu.SemaphoreType.DMA,
          pltpu.SemaphoreType.DMA,
      ],
  )
  def sc_kernel(x_hbm, ei_hbm, out_hbm,
                cidx0, cidx1, ridx0, ridx1, buf0, buf1, buf2, buf3, acc,
                sem0, sem1, sem2, sem3, isem):
    c = lax.axis_index("c")
    s = lax.axis_index("s")
    w = c * NS + s

    cidxs = (cidx0, cidx1)
    ridxs = (ridx0, ridx1)
    bufs = (buf0, buf1, buf2, buf3)
    sems = (sem0, sem1, sem2, sem3)

    def fire_idx(h, par):
      pltpu.async_copy(ei_hbm.at[1, w, h], cidxs[par], isem)
      pltpu.async_copy(ei_hbm.at[0, w, h], ridxs[par], isem)

    def drain_idx(h, par):
      pltpu.make_async_copy(ei_hbm.at[1, w, h], cidxs[par], isem).wait()
      pltpu.make_async_copy(ei_hbm.at[0, w, h], ridxs[par], isem).wait()

    fire_idx(0, 0)

    def zero_row(i, carry):
      for j in range(D // 16):
        buf0[i, pl.ds(j * 16, 16)] = jnp.zeros((16,), jnp.float32)
      return carry
    lax.fori_loop(0, ZCHUNK, zero_row, 0)
    base = s * SLAB
    for j in range(SLAB // ZCHUNK):
      pltpu.sync_copy(buf0.at[pl.ds(0, ZCHUNK)],
                      acc.at[pl.ds(base + j * ZCHUNK, ZCHUNK)])

    @pl.when(s == NS - 1)
    def _():
      pltpu.sync_copy(buf0.at[pl.ds(0, 16)],
                      acc.at[pl.ds(NS * SLAB, 16)])

    drain_idx(0, 0)
    plsc.subcore_barrier()

    def issue(v):
      par = (v // SK) % 2
      pltpu.async_copy(x_hbm.at[cidxs[par].at[v % SK]],
                       bufs[v % NBUF], sems[v % NBUF])

    def step(j, v, last=False):
      b = v % NBUF
      par = (v // SK) % 2
      pltpu.make_async_copy(x_hbm.at[cidxs[par].at[v % SK]],
                            bufs[b], sems[b]).wait()
      if not last:
        if v % SK == 0:
          fire_idx(j * (PERIOD // SK) + v // SK + 1, (par + 1) % 2)
        if v % SK == 2:
          drain_idx(j * (PERIOD // SK) + v // SK + 1, (par + 1) % 2)
        issue(v + 3)
      elif j * PERIOD + v + 3 < K:
        issue(v + 3)
      pltpu.sync_copy(bufs[b], acc.at[ridxs[par].at[v % SK]], add=True)

    for v in range(3):
      issue(v)

    def body(j, carry):
      for v in range(PERIOD):
        step(j, v)
      return carry
    lax.fori_loop(0, K // PERIOD, body, 0)
    for v in range(PERIOD, PERIOD + SK):
      step(K // PERIOD - 1, v, last=True)

    plsc.subcore_barrier()
    pltpu.sync_copy(acc.at[pl.ds(base, SLAB)],
                    out_hbm.at[c, pl.ds(base, SLAB)])

    @pl.when(s == NS - 1)
    def _():
      pltpu.sync_copy(acc.at[pl.ds(NS * SLAB, 16)],
                      out_hbm.at[c, pl.ds(NS * SLAB, 16)])

  return sc_kernel(x, ei5)


def _tc_add(a, b):
  def add_kernel(a_ref, b_ref, o_ref):
    o_ref[...] = a_ref[...] + b_ref[...]

  block = pl.BlockSpec((1000, D), lambda i: (i, 0))
  return pl.pallas_call(
      add_kernel,
      grid=(N_NODES // 1000,),
      in_specs=[block, block],
      out_specs=block,
      out_shape=jax.ShapeDtypeStruct((N_NODES, D), jnp.float32),
  )(a, b)


@jax.jit
def kernel(x, edge_index):
  ei5 = edge_index.astype(jnp.int32).reshape(2, NW, STAGES, SK, CHUNK)
  partials = _sc_partial_sums(x, ei5)
  return _tc_add(partials[0], partials[1])


# speedup vs baseline: 1.2016x; 1.0262x over previous
import functools

import jax
import jax.numpy as jnp
from jax import lax
from jax.experimental import pallas as pl
from jax.experimental.pallas import tpu as pltpu
from jax.experimental.pallas import tpu_sc as plsc

N_NODES = 10000
D = 128
N_EDGES = 320000

NC = 2
NS = 16
NW = NC * NS
E_PER_W = N_EDGES // NW
CHUNK = 80
K = E_PER_W // CHUNK
SK = 5
STAGES = K // SK
NBUF = 4
PERIOD = 20
SLAB = 624
ZCHUNK = 78


def _sc_partial_sums(x, ei5):
  mesh = plsc.VectorSubcoreMesh(core_axis_name="c", subcore_axis_name="s")

  @functools.partial(
      pl.kernel,
      out_type=jax.ShapeDtypeStruct((NC, N_NODES, D), jnp.float32),
      mesh=mesh,
      scratch_types=[
          pltpu.VMEM((SK, CHUNK), jnp.int32),
          pltpu.VMEM((SK, CHUNK), jnp.int32),
          pltpu.VMEM((SK, CHUNK), jnp.int32),
          pltpu.VMEM((SK, CHUNK), jnp.int32),
          pltpu.VMEM((CHUNK, D), jnp.float32),
          pltpu.VMEM((CHUNK, D), jnp.float32),
          pltpu.VMEM((CHUNK, D), jnp.float32),
          pltpu.VMEM((CHUNK, D), jnp.float32),
          pltpu.VMEM_SHARED((N_NODES, D), jnp.float32),
          pltpu.SemaphoreType.DMA,
          pltpu.SemaphoreType.DMA,
          pltpu.SemaphoreType.DMA,
          pltpu.SemaphoreType.DMA,
          pltpu.SemaphoreType.DMA,
      ],
  )
  def sc_kernel(x_hbm, ei_hbm, out_hbm,
                cidx0, cidx1, ridx0, ridx1, buf0, buf1, buf2, buf3, acc,
                sem0, sem1, sem2, sem3, isem):
    c = lax.axis_index("c")
    s = lax.axis_index("s")
    w = c * NS + s

    cidxs = (cidx0, cidx1)
    ridxs = (ridx0, ridx1)
    bufs = (buf0, buf1, buf2, buf3)
    sems = (sem0, sem1, sem2, sem3)

    def fire_idx(h, par):
      pltpu.async_copy(ei_hbm.at[1, w, h], cidxs[par], isem)
      pltpu.async_copy(ei_hbm.at[0, w, h], ridxs[par], isem)

    def drain_idx(h, par):
      pltpu.make_async_copy(ei_hbm.at[1, w, h], cidxs[par], isem).wait()
      pltpu.make_async_copy(ei_hbm.at[0, w, h], ridxs[par], isem).wait()

    fire_idx(0, 0)

    def zero_row(i, carry):
      for j in range(D // 16):
        buf0[i, pl.ds(j * 16, 16)] = jnp.zeros((16,), jnp.float32)
      return carry
    lax.fori_loop(0, ZCHUNK, zero_row, 0)
    base = s * SLAB
    for j in range(SLAB // ZCHUNK):
      pltpu.async_copy(buf0.at[pl.ds(0, ZCHUNK)],
                      acc.at[pl.ds(base + j * ZCHUNK, ZCHUNK)], sem3)

    @pl.when(s == NS - 1)
    def _():
      pltpu.sync_copy(buf0.at[pl.ds(0, 16)],
                      acc.at[pl.ds(NS * SLAB, 16)])

    drain_idx(0, 0)
    for j in range(SLAB // ZCHUNK):
      pltpu.make_async_copy(buf0.at[pl.ds(0, ZCHUNK)],
                            acc.at[pl.ds(base + j * ZCHUNK, ZCHUNK)],
                            sem3).wait()
    plsc.subcore_barrier()

    def issue(v):
      par = (v // SK) % 2
      pltpu.async_copy(x_hbm.at[cidxs[par].at[v % SK]],
                       bufs[v % NBUF], sems[v % NBUF])

    def step(j, v, last=False):
      b = v % NBUF
      par = (v // SK) % 2
      pltpu.make_async_copy(x_hbm.at[cidxs[par].at[v % SK]],
                            bufs[b], sems[b]).wait()
      if not last:
        if v % SK == 0:
          fire_idx(j * (PERIOD // SK) + v // SK + 1, (par + 1) % 2)
        if v % SK == 2:
          drain_idx(j * (PERIOD // SK) + v // SK + 1, (par + 1) % 2)
        issue(v + 3)
      elif j * PERIOD + v + 3 < K:
        issue(v + 3)
      pltpu.sync_copy(bufs[b], acc.at[ridxs[par].at[v % SK]], add=True)

    for v in range(3):
      issue(v)

    def body(j, carry):
      for v in range(PERIOD):
        step(j, v)
      return carry
    lax.fori_loop(0, K // PERIOD, body, 0)
    for v in range(PERIOD, PERIOD + SK):
      step(K // PERIOD - 1, v, last=True)

    plsc.subcore_barrier()
    pltpu.sync_copy(acc.at[pl.ds(base, SLAB)],
                    out_hbm.at[c, pl.ds(base, SLAB)])

    @pl.when(s == NS - 1)
    def _():
      pltpu.sync_copy(acc.at[pl.ds(NS * SLAB, 16)],
                      out_hbm.at[c, pl.ds(NS * SLAB, 16)])

  return sc_kernel(x, ei5)


def _tc_add(a, b):
  def add_kernel(a_ref, b_ref, o_ref):
    o_ref[...] = a_ref[...] + b_ref[...]

  return pl.pallas_call(
      add_kernel,
      out_shape=jax.ShapeDtypeStruct((N_NODES, D), jnp.float32),
  )(a, b)


@jax.jit
def kernel(x, edge_index):
  ei5 = edge_index.astype(jnp.int32).reshape(2, NW, STAGES, SK, CHUNK)
  partials = _sc_partial_sums(x, ei5)
  return _tc_add(partials[0], partials[1])
